# Initial kernel scaffold; baseline (speedup 1.0000x reference)
import functools

import jax
import jax.numpy as jnp
from jax import lax
from jax.experimental import pallas as pl
from jax.experimental.pallas import tpu as pltpu
from jax.experimental.pallas import tpu_sc as plsc

BATCH = 16384
FIELDS = 26
NC = 2
NS = 16
NW = NC * NS
BPW = BATCH // NW
IPW = BPW * FIELDS
LANES = 16


def _body(emb_ref, x_ref, bias_ref, out_ref, xv, gv, accv, bv, sem):
    wid = lax.axis_index("s") * NC + lax.axis_index("c")
    base = wid * BPW

    pltpu.sync_copy(x_ref.at[pl.ds(base * FIELDS, IPW)], xv)
    pltpu.sync_copy(bias_ref, bv)

    pltpu.async_copy(emb_ref.at[xv], gv, sem).wait()

    bvec = bv[...]
    lane_off = lax.iota(jnp.int32, LANES) * FIELDS

    def chunk(c, _):
        flat = c * (LANES * FIELDS)
        acc = bvec
        for f in range(FIELDS):
            acc = acc + plsc.load_gather(gv, [lane_off + (flat + f)])
        accv[pl.ds(c * LANES, LANES)] = acc
        return _

    lax.fori_loop(0, BPW // LANES, chunk, None)
    pltpu.sync_copy(accv, out_ref.at[pl.ds(base, BPW)])


@functools.partial(jax.jit, static_argnames=())
def kernel(x, emb, bias):
    x_flat = x.reshape(-1).astype(jnp.int32)
    emb_flat = emb.reshape(-1)
    bias16 = jnp.broadcast_to(bias, (LANES,))

    call = pl.kernel(
        _body,
        out_type=jax.ShapeDtypeStruct((BATCH,), jnp.float32),
        mesh=plsc.VectorSubcoreMesh(core_axis_name="c", subcore_axis_name="s"),
        scratch_types=[
            pltpu.VMEM((IPW,), jnp.int32),
            pltpu.VMEM((IPW,), jnp.float32),
            pltpu.VMEM((BPW,), jnp.float32),
            pltpu.VMEM((LANES,), jnp.float32),
            pltpu.SemaphoreType.DMA,
        ],
    )
    out = call(emb_flat, x_flat, bias16)
    return out.reshape(BATCH, 1)


# speedup vs baseline: 1.4730x; 1.4730x over previous
import functools

import jax
import jax.numpy as jnp
from jax import lax
from jax.experimental import pallas as pl
from jax.experimental.pallas import tpu as pltpu
from jax.experimental.pallas import tpu_sc as plsc

BATCH = 16384
FIELDS = 26
NC = 2
NS = 16
NW = NC * NS
BPW = BATCH // NW
IPW = BPW * FIELDS
LANES = 16


def _body(emb_ref, xt_ref, bias_ref, out_ref, xv, gv, accv, bv, sem, stage_sem):
    wid = lax.axis_index("s") * NC + lax.axis_index("c")
    base = wid * BPW

    stage = [
        pltpu.async_copy(
            xt_ref.at[pl.ds(f * BATCH + base, BPW)],
            xv.at[pl.ds(f * BPW, BPW)],
            stage_sem,
        )
        for f in range(FIELDS)
    ]
    pltpu.sync_copy(bias_ref, bv)
    for h in stage:
        h.wait()

    pltpu.async_copy(emb_ref.at[xv], gv, sem).wait()

    bvec = bv[...]

    def chunk(c, _):
        off = c * LANES
        acc = bvec
        for f in range(FIELDS):
            acc = acc + gv[pl.ds(f * BPW + off, LANES)]
        accv[pl.ds(off, LANES)] = acc
        return _

    lax.fori_loop(0, BPW // LANES, chunk, None)
    pltpu.sync_copy(accv, out_ref.at[pl.ds(base, BPW)])


@functools.partial(jax.jit, static_argnames=())
def kernel(x, emb, bias):
    xt = x.T.astype(jnp.int32).reshape(-1)
    emb_flat = emb.reshape(-1)
    bias16 = jnp.broadcast_to(bias, (LANES,))

    call = pl.kernel(
        _body,
        out_type=jax.ShapeDtypeStruct((BATCH,), jnp.float32),
        mesh=plsc.VectorSubcoreMesh(core_axis_name="c", subcore_axis_name="s"),
        scratch_types=[
            pltpu.VMEM((IPW,), jnp.int32),
            pltpu.VMEM((IPW,), jnp.float32),
            pltpu.VMEM((BPW,), jnp.float32),
            pltpu.VMEM((LANES,), jnp.float32),
            pltpu.SemaphoreType.DMA,
            pltpu.SemaphoreType.DMA,
        ],
    )
    out = call(emb_flat, xt, bias16)
    return out.reshape(BATCH, 1)


# speedup vs baseline: 2.5140x; 1.7067x over previous
import functools

import jax
import jax.numpy as jnp
from jax import lax
from jax.experimental import pallas as pl
from jax.experimental.pallas import tpu as pltpu
from jax.experimental.pallas import tpu_sc as plsc

BATCH = 16384
FIELDS = 26
VOCAB = 1000000
VOCAB_PAD = 977 * 1024
NC = 2
NS = 16
NW = NC * NS
BPW = BATCH // NW
IPW = BPW * FIELDS
LANES = 16


def _body(emb_ref, xt_ref, bias_ref, out_ref, xv, gv, accv, bv, sem, stage_sem):
    wid = lax.axis_index("s") * NC + lax.axis_index("c")
    base = wid * BPW

    stage = [
        pltpu.async_copy(
            xt_ref.at[pl.ds(f * BATCH + base, BPW)],
            xv.at[pl.ds(f * BPW, BPW)],
            stage_sem,
        )
        for f in range(FIELDS)
    ]
    pltpu.sync_copy(bias_ref, bv)
    for h in stage:
        h.wait()

    pltpu.async_copy(emb_ref.at[xv], gv, sem).wait()

    bvec = bv[...]

    def chunk(c, _):
        off = c * LANES
        acc = bvec
        for f in range(FIELDS):
            acc = acc + gv[pl.ds(f * BPW + off, LANES)]
        accv[pl.ds(off, LANES)] = acc
        return _

    lax.fori_loop(0, BPW // LANES, chunk, None)
    pltpu.sync_copy(accv, out_ref.at[pl.ds(base, BPW)])


@functools.partial(jax.jit, static_argnames=())
def kernel(x, emb, bias):
    xt = x.T.astype(jnp.int32).reshape(-1)
    emb_flat = jnp.concatenate(
        [emb, jnp.zeros((VOCAB_PAD - VOCAB, 1), jnp.float32)], axis=0
    ).reshape(-1)
    bias16 = jnp.broadcast_to(bias, (LANES,))

    call = pl.kernel(
        _body,
        out_type=jax.ShapeDtypeStruct((BATCH,), jnp.float32),
        mesh=plsc.VectorSubcoreMesh(core_axis_name="c", subcore_axis_name="s"),
        scratch_types=[
            pltpu.VMEM((IPW,), jnp.int32),
            pltpu.VMEM((IPW,), jnp.float32),
            pltpu.VMEM((BPW,), jnp.float32),
            pltpu.VMEM((LANES,), jnp.float32),
            pltpu.SemaphoreType.DMA,
            pltpu.SemaphoreType.DMA,
        ],
    )
    out = call(emb_flat, xt, bias16)
    return out.reshape(BATCH, 1)


# speedup vs baseline: 2.5633x; 1.0196x over previous
import functools

import jax
import jax.numpy as jnp
from jax import lax
from jax.experimental import pallas as pl
from jax.experimental.pallas import tpu as pltpu
from jax.experimental.pallas import tpu_sc as plsc

BATCH = 16384
FIELDS = 26
VOCAB = 1000000
VOCAB_PAD = 977 * 1024
NC = 2
NS = 16
NW = NC * NS
BPW = BATCH // NW
IPW = BPW * FIELDS
LANES = 16
GF = 2
NG = FIELDS // GF
GW = GF * BPW


def _body(emb_ref, xt_ref, bias_ref, out_ref,
          xv, gv, accv, bv, ss0, ss1, gs0, gs1):
    wid = lax.axis_index("s") * NC + lax.axis_index("c")
    base = wid * BPW
    ssem = (ss0, ss1)
    gsem = (gs0, gs1)

    def fire_stage(g):
        return [
            pltpu.async_copy(
                xt_ref.at[pl.ds((g * GF + j) * BATCH + base, BPW)],
                xv.at[pl.ds((g * GF + j) * BPW, BPW)],
                ssem[g % 2],
            )
            for j in range(GF)
        ]

    def fire_gather(g):
        return pltpu.async_copy(
            emb_ref.at[xv.at[pl.ds(g * GW, GW)]],
            gv.at[pl.ds(g * GW, GW)],
            gsem[g % 2],
        )

    stage_h = {0: fire_stage(0), 1: fire_stage(1)}
    pltpu.sync_copy(bias_ref, bv)
    for h in stage_h[0]:
        h.wait()
    gather_h = {0: fire_gather(0)}
    bvec = bv[...]

    for g in range(NG):
        if g + 2 < NG:
            stage_h[g + 2] = fire_stage(g + 2)
        if g + 1 < NG:
            for h in stage_h[g + 1]:
                h.wait()
            gather_h[g + 1] = fire_gather(g + 1)
        gather_h[g].wait()

        def chunk(c, _, g=g):
            off = c * LANES
            acc = bvec if g == 0 else accv[pl.ds(off, LANES)]
            for j in range(GF):
                acc = acc + gv[pl.ds((g * GF + j) * BPW + off, LANES)]
            accv[pl.ds(off, LANES)] = acc
            return _

        lax.fori_loop(0, BPW // LANES, chunk, None)

    pltpu.sync_copy(accv, out_ref.at[pl.ds(base, BPW)])


@functools.partial(jax.jit, static_argnames=())
def kernel(x, emb, bias):
    xt = x.T.astype(jnp.int32).reshape(-1)
    emb_flat = jnp.pad(emb, ((0, VOCAB_PAD - VOCAB), (0, 0))).reshape(-1)
    bias16 = jnp.broadcast_to(bias, (LANES,))

    call = pl.kernel(
        _body,
        out_type=jax.ShapeDtypeStruct((BATCH,), jnp.float32),
        mesh=plsc.VectorSubcoreMesh(core_axis_name="c", subcore_axis_name="s"),
        scratch_types=[
            pltpu.VMEM((IPW,), jnp.int32),
            pltpu.VMEM((IPW,), jnp.float32),
            pltpu.VMEM((BPW,), jnp.float32),
            pltpu.VMEM((LANES,), jnp.float32),
            pltpu.SemaphoreType.DMA,
            pltpu.SemaphoreType.DMA,
            pltpu.SemaphoreType.DMA,
            pltpu.SemaphoreType.DMA,
        ],
    )
    out = call(emb_flat, xt, bias16)
    return out.reshape(BATCH, 1)


# speedup vs baseline: 2.5722x; 1.0035x over previous
import functools

import jax
import jax.numpy as jnp
from jax import lax
from jax.experimental import pallas as pl
from jax.experimental.pallas import tpu as pltpu
from jax.experimental.pallas import tpu_sc as plsc

BATCH = 16384
FIELDS = 26
VOCAB = 1000000
VOCAB_PAD = 977 * 1024
NC = 2
NS = 16
NW = NC * NS
BPW = BATCH // NW
IPW = BPW * FIELDS
LANES = 16
GF = 2
NG = FIELDS // GF
GW = GF * BPW


def _body(emb_ref, xt_ref, bias_ref, out_ref,
          xv, gv, accv, bv, ss0, ss1, gs0, gs1):
    wid = lax.axis_index("s") * NC + lax.axis_index("c")
    base = wid * BPW
    ssem = (ss0, ss1)
    gsem = (gs0, gs1)

    def fire_stage(g):
        return [
            pltpu.async_copy(
                xt_ref.at[pl.ds((g * GF + j) * BATCH + base, BPW)],
                xv.at[pl.ds((g * GF + j) * BPW, BPW)],
                ssem[g % 2],
            )
            for j in range(GF)
        ]

    def fire_gather(g):
        return pltpu.async_copy(
            emb_ref.at[xv.at[pl.ds(g * GW, GW)]],
            gv.at[pl.ds(g * GW, GW)],
            gsem[g % 2],
        )

    stage_h = {g: fire_stage(g) for g in range(min(4, NG))}
    pltpu.sync_copy(bias_ref, bv)
    gather_h = {}
    for g in (0, 1):
        for h in stage_h[g]:
            h.wait()
        gather_h[g] = fire_gather(g)
    bvec = bv[...]

    for g in range(NG):
        gather_h[g].wait()
        if g + 2 < NG:
            for h in stage_h[g + 2]:
                h.wait()
            gather_h[g + 2] = fire_gather(g + 2)
        if g + 4 < NG:
            stage_h[g + 4] = fire_stage(g + 4)

        def chunk(c, _, g=g):
            off = c * LANES
            acc = bvec if g == 0 else accv[pl.ds(off, LANES)]
            for j in range(GF):
                acc = acc + gv[pl.ds((g * GF + j) * BPW + off, LANES)]
            accv[pl.ds(off, LANES)] = acc
            return _

        lax.fori_loop(0, BPW // LANES, chunk, None)

    pltpu.sync_copy(accv, out_ref.at[pl.ds(base, BPW)])


@functools.partial(jax.jit, static_argnames=())
def kernel(x, emb, bias):
    xt = x.T.astype(jnp.int32).reshape(-1)
    emb_flat = jnp.pad(emb, ((0, VOCAB_PAD - VOCAB), (0, 0))).reshape(-1)
    bias16 = jnp.broadcast_to(bias, (LANES,))

    call = pl.kernel(
        _body,
        out_type=jax.ShapeDtypeStruct((BATCH,), jnp.float32),
        mesh=plsc.VectorSubcoreMesh(core_axis_name="c", subcore_axis_name="s"),
        scratch_types=[
            pltpu.VMEM((IPW,), jnp.int32),
            pltpu.VMEM((IPW,), jnp.float32),
            pltpu.VMEM((BPW,), jnp.float32),
            pltpu.VMEM((LANES,), jnp.float32),
            pltpu.SemaphoreType.DMA,
            pltpu.SemaphoreType.DMA,
            pltpu.SemaphoreType.DMA,
            pltpu.SemaphoreType.DMA,
        ],
    )
    out = call(emb_flat, xt, bias16)
    return out.reshape(BATCH, 1)


# speedup vs baseline: 2.7447x; 1.0671x over previous
import functools

import jax
import jax.numpy as jnp
from jax import lax
from jax.experimental import pallas as pl
from jax.experimental.pallas import tpu as pltpu
from jax.experimental.pallas import tpu_sc as plsc

BATCH = 16384
FIELDS = 26
VOCAB = 1000000
VOCAB_PAD = 977 * 1024
NC = 2
NS = 16
NW = NC * NS
BPW = BATCH // NW
IPW = BPW * FIELDS
LANES = 16
GF = 2
NG = FIELDS // GF
GW = GF * BPW


def _body(emb_ref, xt_ref, bias_ref, out_ref,
          xv, gv, accv, bv, ss0, ss1, gs0, gs1):
    wid = lax.axis_index("s") * NC + lax.axis_index("c")
    base = wid * BPW
    ssem = (ss0, ss1)
    gsem = (gs0, gs1)

    def fire_stage(g):
        return [
            pltpu.async_copy(
                xt_ref.at[g * GF + j, pl.ds(base, BPW)],
                xv.at[pl.ds((g * GF + j) * BPW, BPW)],
                ssem[g % 2],
            )
            for j in range(GF)
        ]

    def fire_gather(g):
        return pltpu.async_copy(
            emb_ref.at[xv.at[pl.ds(g * GW, GW)]],
            gv.at[pl.ds(g * GW, GW)],
            gsem[g % 2],
        )

    stage_h = {g: fire_stage(g) for g in range(min(4, NG))}
    pltpu.sync_copy(bias_ref, bv)
    gather_h = {}
    for g in (0, 1):
        for h in stage_h[g]:
            h.wait()
        gather_h[g] = fire_gather(g)
    bvec = bv[...]

    for g in range(NG):
        gather_h[g].wait()
        if g + 2 < NG:
            for h in stage_h[g + 2]:
                h.wait()
            gather_h[g + 2] = fire_gather(g + 2)
        if g + 4 < NG:
            stage_h[g + 4] = fire_stage(g + 4)

        def chunk(c, _, g=g):
            off = c * LANES
            acc = bvec if g == 0 else accv[pl.ds(off, LANES)]
            for j in range(GF):
                acc = acc + gv[pl.ds((g * GF + j) * BPW + off, LANES)]
            accv[pl.ds(off, LANES)] = acc
            return _

        lax.fori_loop(0, BPW // LANES, chunk, None)

    pltpu.sync_copy(accv, out_ref.at[pl.ds(base, BPW)])


@functools.partial(jax.jit, static_argnames=())
def kernel(x, emb, bias):
    xt = x.T.astype(jnp.int32)
    emb_flat = jnp.pad(emb, ((0, VOCAB_PAD - VOCAB), (0, 0))).reshape(-1)
    bias16 = jnp.broadcast_to(bias, (LANES,))

    call = pl.kernel(
        _body,
        out_type=jax.ShapeDtypeStruct((BATCH,), jnp.float32),
        mesh=plsc.VectorSubcoreMesh(core_axis_name="c", subcore_axis_name="s"),
        scratch_types=[
            pltpu.VMEM((IPW,), jnp.int32),
            pltpu.VMEM((IPW,), jnp.float32),
            pltpu.VMEM((BPW,), jnp.float32),
            pltpu.VMEM((LANES,), jnp.float32),
            pltpu.SemaphoreType.DMA,
            pltpu.SemaphoreType.DMA,
            pltpu.SemaphoreType.DMA,
            pltpu.SemaphoreType.DMA,
        ],
    )
    out = call(emb_flat, xt, bias16)
    return out.reshape(BATCH, 1)


# speedup vs baseline: 3.3205x; 1.2098x over previous
import functools

import jax
import jax.numpy as jnp
from jax import lax
from jax.experimental import pallas as pl
from jax.experimental.pallas import tpu as pltpu
from jax.experimental.pallas import tpu_sc as plsc

BATCH = 16384
FIELDS = 26
VOCAB = 1000000
VOCAB_PAD = 977 * 1024
NC = 2
NS = 16
NW = NC * NS
BPW = BATCH // NW
IPW = BPW * FIELDS
LANES = 16
GF = 2
NG = FIELDS // GF
GW = GF * BPW


def _body(emb_ref, xt_ref, bias_ref, out_ref,
          xv, gv, accv, bv, spm, ss0, ss1, gs0, gs1, ts):
    wid = lax.axis_index("s") * NC + lax.axis_index("c")
    base = wid * BPW
    ssem = (ss0, ss1)
    gsem = (gs0, gs1)

    sid = lax.axis_index("s")
    tchunk = 61 * 1024
    tload = [
        pltpu.async_copy(
            emb_ref.at[pl.ds(sid * tchunk, tchunk)],
            spm.at[pl.ds(sid * tchunk, tchunk)],
            ts,
        ),
        pltpu.async_copy(
            emb_ref.at[pl.ds(NS * tchunk, 1024)],
            spm.at[pl.ds(NS * tchunk, 1024)],
            ts,
        ),
    ]

    def fire_stage(g):
        return [
            pltpu.async_copy(
                xt_ref.at[g * GF + j, pl.ds(base, BPW)],
                xv.at[pl.ds((g * GF + j) * BPW, BPW)],
                ssem[g % 2],
            )
            for j in range(GF)
        ]

    def fire_gather(g):
        return pltpu.async_copy(
            spm.at[xv.at[pl.ds(g * GW, GW)]],
            gv.at[pl.ds(g * GW, GW)],
            gsem[g % 2],
        )

    stage_h = {g: fire_stage(g) for g in range(min(4, NG))}
    pltpu.sync_copy(bias_ref, bv)
    for h in tload:
        h.wait()
    plsc.subcore_barrier()
    gather_h = {}
    for g in (0, 1):
        for h in stage_h[g]:
            h.wait()
        gather_h[g] = fire_gather(g)
    bvec = bv[...]

    for g in range(NG):
        gather_h[g].wait()
        if g + 2 < NG:
            for h in stage_h[g + 2]:
                h.wait()
            gather_h[g + 2] = fire_gather(g + 2)
        if g + 4 < NG:
            stage_h[g + 4] = fire_stage(g + 4)

        def chunk(c, _, g=g):
            off = c * LANES
            acc = bvec if g == 0 else accv[pl.ds(off, LANES)]
            for j in range(GF):
                acc = acc + gv[pl.ds((g * GF + j) * BPW + off, LANES)]
            accv[pl.ds(off, LANES)] = acc
            return _

        lax.fori_loop(0, BPW // LANES, chunk, None)

    pltpu.sync_copy(accv, out_ref.at[pl.ds(base, BPW)])


@functools.partial(jax.jit, static_argnames=())
def kernel(x, emb, bias):
    xt = x.T.astype(jnp.int32)
    emb_flat = jnp.pad(emb, ((0, VOCAB_PAD - VOCAB), (0, 0))).reshape(-1)
    bias16 = jnp.broadcast_to(bias, (LANES,))

    call = pl.kernel(
        _body,
        out_type=jax.ShapeDtypeStruct((BATCH,), jnp.float32),
        mesh=plsc.VectorSubcoreMesh(core_axis_name="c", subcore_axis_name="s"),
        scratch_types=[
            pltpu.VMEM((IPW,), jnp.int32),
            pltpu.VMEM((IPW,), jnp.float32),
            pltpu.VMEM((BPW,), jnp.float32),
            pltpu.VMEM((LANES,), jnp.float32),
            pltpu.VMEM_SHARED((VOCAB_PAD,), jnp.float32),
            pltpu.SemaphoreType.DMA,
            pltpu.SemaphoreType.DMA,
            pltpu.SemaphoreType.DMA,
            pltpu.SemaphoreType.DMA,
            pltpu.SemaphoreType.DMA,
        ],
    )
    out = call(emb_flat, xt, bias16)
    return out.reshape(BATCH, 1)


# speedup vs baseline: 3.5033x; 1.0551x over previous
import functools

import jax
import jax.numpy as jnp
from jax import lax
from jax.experimental import pallas as pl
from jax.experimental.pallas import tpu as pltpu
from jax.experimental.pallas import tpu_sc as plsc

BATCH = 16384
FIELDS = 26
VOCAB = 1000000
VOCAB_PAD = 977 * 1024
NC = 2
NS = 16
NW = NC * NS
BPW = BATCH // NW
IPW = BPW * FIELDS
LANES = 16
GF = 2
NG = FIELDS // GF
GW = GF * BPW


def _body(emb_ref, tail_ref, xt_ref, bias_ref, out_ref,
          xv, gv, accv, bv, spm, ss0, ss1, gs0, gs1, ts):
    wid = lax.axis_index("s") * NC + lax.axis_index("c")
    base = wid * BPW
    ssem = (ss0, ss1)
    gsem = (gs0, gs1)

    sid = lax.axis_index("s")
    tchunk = 61 * 1024
    tload = [
        pltpu.async_copy(
            emb_ref.at[pl.ds(sid * tchunk, tchunk)],
            spm.at[pl.ds(sid * tchunk, tchunk)],
            ts,
        ),
        pltpu.async_copy(
            tail_ref,
            spm.at[pl.ds(NS * tchunk, 1024)],
            ts,
        ),
    ]

    def fire_stage(g):
        return [
            pltpu.async_copy(
                xt_ref.at[g * GF + j, pl.ds(base, BPW)],
                xv.at[pl.ds((g * GF + j) * BPW, BPW)],
                ssem[g % 2],
            )
            for j in range(GF)
        ]

    def fire_gather(g):
        return pltpu.async_copy(
            spm.at[xv.at[pl.ds(g * GW, GW)]],
            gv.at[pl.ds(g * GW, GW)],
            gsem[g % 2],
        )

    stage_h = {g: fire_stage(g) for g in range(min(4, NG))}
    pltpu.sync_copy(bias_ref, bv)
    for h in tload:
        h.wait()
    plsc.subcore_barrier()
    gather_h = {}
    for g in (0, 1):
        for h in stage_h[g]:
            h.wait()
        gather_h[g] = fire_gather(g)
    bvec = bv[...]

    for g in range(NG):
        gather_h[g].wait()
        if g + 2 < NG:
            for h in stage_h[g + 2]:
                h.wait()
            gather_h[g + 2] = fire_gather(g + 2)
        if g + 4 < NG:
            stage_h[g + 4] = fire_stage(g + 4)

        def chunk(c, _, g=g):
            off = c * LANES
            acc = bvec if g == 0 else accv[pl.ds(off, LANES)]
            for j in range(GF):
                acc = acc + gv[pl.ds((g * GF + j) * BPW + off, LANES)]
            accv[pl.ds(off, LANES)] = acc
            return _

        lax.fori_loop(0, BPW // LANES, chunk, None)

    pltpu.sync_copy(accv, out_ref.at[pl.ds(base, BPW)])


@functools.partial(jax.jit, static_argnames=())
def kernel(x, emb, bias):
    xt = x.T.astype(jnp.int32)
    head = 16 * 61 * 1024
    emb_head = emb[:head].reshape(-1)
    emb_tail = jnp.pad(
        emb[head:], ((0, VOCAB_PAD - VOCAB), (0, 0))
    ).reshape(-1)
    bias16 = jnp.broadcast_to(bias, (LANES,))

    call = pl.kernel(
        _body,
        out_type=jax.ShapeDtypeStruct((BATCH,), jnp.float32),
        mesh=plsc.VectorSubcoreMesh(core_axis_name="c", subcore_axis_name="s"),
        scratch_types=[
            pltpu.VMEM((IPW,), jnp.int32),
            pltpu.VMEM((IPW,), jnp.float32),
            pltpu.VMEM((BPW,), jnp.float32),
            pltpu.VMEM((LANES,), jnp.float32),
            pltpu.VMEM_SHARED((VOCAB_PAD,), jnp.float32),
            pltpu.SemaphoreType.DMA,
            pltpu.SemaphoreType.DMA,
            pltpu.SemaphoreType.DMA,
            pltpu.SemaphoreType.DMA,
            pltpu.SemaphoreType.DMA,
        ],
    )
    out = call(emb_head, emb_tail, xt, bias16)
    return out.reshape(BATCH, 1)
